# baseline (speedup 1.0000x reference)
import jax
import jax.numpy as jnp
from jax import lax
from jax.experimental import pallas as pl
from jax.experimental.pallas import tpu as pltpu
from jax.experimental.pallas import tpu_sc as plsc

VOCAB = 100000
EMBED = 128
B, T = 16, 2048
ROWS = B * T
NS = 16
TW0 = 96
TW1 = (T - NS * TW0) // NS
T_SPLIT = NS * TW0
NB = 4
LOOKAHEAD = 2


def _body(idx0_hbm, idx1_hbm, tok_hbm, pos_hbm, out_hbm,
          idx0_v, idx1_v, pos_v,
          r0, r1, r2, r3, g0, g1, g2, g3, s0, s1, s2, s3, psem):
    rows = [r0, r1, r2, r3]
    gsem = [g0, g1, g2, g3]
    ssem = [s0, s1, s2, s3]
    cid = lax.axis_index("c")
    sid = lax.axis_index("s")

    def pipeline(idx_hbm, idx_v, tw, t0):
        c_pos = pltpu.async_copy(
            pos_hbm.at[pl.ds(t0, tw)], pos_v.at[pl.ds(0, tw)], psem)
        pltpu.sync_copy(idx_hbm.at[sid], idx_v)
        c_pos.wait()

        g = {}
        s = {}

        def start_gather(b):
            buf = b % NB
            g[b] = pltpu.async_copy(
                tok_hbm.at[idx_v.at[b]], rows[buf].at[pl.ds(0, tw)], gsem[buf])

        for b in range(LOOKAHEAD):
            start_gather(b)

        for b in range(B):
            buf = b % NB
            nb = b + LOOKAHEAD
            if nb < B:
                pb = nb - NB
                if pb >= 0:
                    s[pb].wait()
                start_gather(nb)
            g[b].wait()

            def row_body(r, _, buf=buf):
                for c in range(EMBED // 16):
                    sl = pl.ds(c * 16, 16)
                    plsc.addupdate(rows[buf].at[r, sl], pos_v[r, sl])
                return 0

            lax.fori_loop(0, tw, row_body, 0, unroll=2)

            s[b] = pltpu.async_copy(
                rows[buf].at[pl.ds(0, tw)],
                out_hbm.at[pl.ds(b * T + t0, tw)], ssem[buf])

        for b in range(B - NB, B):
            s[b].wait()

    @pl.when(cid == 0)
    def _():
        pipeline(idx0_hbm, idx0_v, TW0, sid * TW0)

    @pl.when(cid == 1)
    def _():
        pipeline(idx1_hbm, idx1_v, TW1, T_SPLIT + sid * TW1)


def kernel(idx, tok_table, pos_table):
    idx = idx.astype(jnp.int32)
    idx0_r = idx[:, :T_SPLIT].reshape(B, NS, TW0).transpose(1, 0, 2)
    idx1_r = idx[:, T_SPLIT:].reshape(B, NS, TW1).transpose(1, 0, 2)
    mesh = plsc.VectorSubcoreMesh(core_axis_name="c", subcore_axis_name="s")
    out = pl.kernel(
        _body,
        mesh=mesh,
        out_type=jax.ShapeDtypeStruct((ROWS, EMBED), jnp.float32),
        scratch_types=[
            pltpu.VMEM((B, TW0), jnp.int32),
            pltpu.VMEM((B, TW1), jnp.int32),
            pltpu.VMEM((TW0, EMBED), jnp.float32),
        ] + [pltpu.VMEM((TW0, EMBED), jnp.float32)] * NB
          + [pltpu.SemaphoreType.DMA] * (2 * NB + 1),
    )(idx0_r, idx1_r, tok_table, pos_table)
    return out.reshape(B, T, EMBED)


# speedup vs baseline: 1.1917x; 1.1917x over previous
import jax
import jax.numpy as jnp
from jax import lax
from jax.experimental import pallas as pl
from jax.experimental.pallas import tpu as pltpu
from jax.experimental.pallas import tpu_sc as plsc

VOCAB = 100000
EMBED = 128
B, T = 16, 2048
ROWS = B * T
NW = 32
TW = T // NW
CB = 2
CROWS = CB * TW
NCH = B // CB
NB = 4
LOOKAHEAD = 2


def _body(idx_hbm, tok_hbm, pos_hbm, out_hbm,
          idx_v, pos_v,
          r0, r1, r2, r3, g0, g1, g2, g3, s0, s1, s2, s3, psem, isem):
    rows = [r0, r1, r2, r3]
    gsem = [g0, g1, g2, g3]
    ssem = [s0, s1, s2, s3]
    wid = lax.axis_index("s") * 2 + lax.axis_index("c")
    t0 = wid * TW

    c_pos = pltpu.async_copy(pos_hbm.at[pl.ds(t0, TW)], pos_v, psem)
    c_idx = [
        pltpu.async_copy(
            idx_hbm.at[b, pl.ds(t0, TW)],
            idx_v.at[b // CB, pl.ds((b % CB) * TW, TW)], isem)
        for b in range(B)
    ]
    for c in c_idx:
        c.wait()
    c_pos.wait()

    g = {}
    s = {}

    def start_gather(j):
        buf = j % NB
        g[j] = pltpu.async_copy(tok_hbm.at[idx_v.at[j]], rows[buf], gsem[buf])

    for j in range(LOOKAHEAD):
        start_gather(j)

    for j in range(NCH):
        buf = j % NB
        nj = j + LOOKAHEAD
        if nj < NCH:
            pj = nj - NB
            if pj >= 0:
                s[pj][0].wait()
                s[pj][1].wait()
            start_gather(nj)
        g[j].wait()

        def row_body(r, _, buf=buf):
            for c in range(EMBED // 16):
                sl = pl.ds(c * 16, 16)
                v = pos_v[r, sl]
                for k in range(CB):
                    plsc.addupdate(rows[buf].at[k * TW + r, sl], v)
            return 0

        lax.fori_loop(0, TW, row_body, 0, unroll=2)

        s[j] = tuple(
            pltpu.async_copy(
                rows[buf].at[pl.ds(k * TW, TW)],
                out_hbm.at[pl.ds((j * CB + k) * T + t0, TW)],
                ssem[buf])
            for k in range(CB))

    for j in range(NCH - NB, NCH):
        s[j][0].wait()
        s[j][1].wait()


def kernel(idx, tok_table, pos_table):
    mesh = plsc.VectorSubcoreMesh(core_axis_name="c", subcore_axis_name="s")
    out = pl.kernel(
        _body,
        mesh=mesh,
        out_type=jax.ShapeDtypeStruct((ROWS, EMBED), jnp.float32),
        scratch_types=[
            pltpu.VMEM((NCH, CROWS), jnp.int32),
            pltpu.VMEM((TW, EMBED), jnp.float32),
        ] + [pltpu.VMEM((CROWS, EMBED), jnp.float32)] * NB
          + [pltpu.SemaphoreType.DMA] * (2 * NB + 2),
    )(idx.astype(jnp.int32), tok_table, pos_table)
    return out.reshape(B, T, EMBED)


# speedup vs baseline: 1.2337x; 1.0352x over previous
import jax
import jax.numpy as jnp
from jax import lax
from jax.experimental import pallas as pl
from jax.experimental.pallas import tpu as pltpu
from jax.experimental.pallas import tpu_sc as plsc

VOCAB = 100000
EMBED = 128
B, T = 16, 2048
ROWS = B * T
NW = 32
TW = T // NW
CB = 2
CROWS = CB * TW
NCH = B // CB
NB = 6
LOOKAHEAD = 3


def _body(idx_hbm, tok_hbm, pos_hbm, out_hbm,
          idx_v, pos_v,
          r0, r1, r2, r3, r4, r5,
          g0, g1, g2, g3, g4, g5,
          s0, s1, s2, s3, s4, s5, psem, isem):
    rows = [r0, r1, r2, r3, r4, r5]
    gsem = [g0, g1, g2, g3, g4, g5]
    ssem = [s0, s1, s2, s3, s4, s5]
    wid = lax.axis_index("s") * 2 + lax.axis_index("c")
    t0 = wid * TW

    c_pos = pltpu.async_copy(pos_hbm.at[pl.ds(t0, TW)], pos_v, psem)
    c_idx = [
        pltpu.async_copy(
            idx_hbm.at[b, pl.ds(t0, TW)],
            idx_v.at[b // CB, pl.ds((b % CB) * TW, TW)], isem)
        for b in range(B)
    ]

    g = {}
    s = {}

    def start_gather(j):
        buf = j % NB
        c_idx[CB * j].wait()
        c_idx[CB * j + 1].wait()
        g[j] = pltpu.async_copy(tok_hbm.at[idx_v.at[j]], rows[buf], gsem[buf])

    for j in range(LOOKAHEAD):
        start_gather(j)

    for j in range(NCH):
        buf = j % NB
        nj = j + LOOKAHEAD
        if nj < NCH:
            pj = nj - NB
            if pj >= 0:
                s[pj][0].wait()
                s[pj][1].wait()
            start_gather(nj)
        g[j].wait()
        if j == 0:
            c_pos.wait()

        def row_body(r, _, buf=buf):
            for c in range(EMBED // 16):
                sl = pl.ds(c * 16, 16)
                v = pos_v[r, sl]
                for k in range(CB):
                    plsc.addupdate(rows[buf].at[k * TW + r, sl], v)
            return 0

        lax.fori_loop(0, TW, row_body, 0, unroll=2)

        s[j] = tuple(
            pltpu.async_copy(
                rows[buf].at[pl.ds(k * TW, TW)],
                out_hbm.at[pl.ds((j * CB + k) * T + t0, TW)],
                ssem[buf])
            for k in range(CB))

    for j in range(NCH - NB, NCH):
        if j >= 0:
            s[j][0].wait()
            s[j][1].wait()


def kernel(idx, tok_table, pos_table):
    mesh = plsc.VectorSubcoreMesh(core_axis_name="c", subcore_axis_name="s")
    out = pl.kernel(
        _body,
        mesh=mesh,
        out_type=jax.ShapeDtypeStruct((ROWS, EMBED), jnp.float32),
        scratch_types=[
            pltpu.VMEM((NCH, CROWS), jnp.int32),
            pltpu.VMEM((TW, EMBED), jnp.float32),
        ] + [pltpu.VMEM((CROWS, EMBED), jnp.float32)] * NB
          + [pltpu.SemaphoreType.DMA] * (2 * NB + 2),
    )(idx.astype(jnp.int32), tok_table, pos_table)
    return out.reshape(B, T, EMBED)
